# Initial kernel scaffold; baseline (speedup 1.0000x reference)
import jax
import jax.numpy as jnp
from jax.experimental import pallas as pl


def kernel(aspect_logits, opinion_logits, aspect_opinion_relations, explicit_aspects, explicit_opinions, W1, b1, W2, b2, W3, b3):
    raise NotImplementedError("write your pallas kernel here")


# speedup vs baseline: 29.6495x; 29.6495x over previous
import functools

import jax
import jax.numpy as jnp
from jax.experimental import pallas as pl
from jax.experimental.pallas import tpu as pltpu

B, S = 8, 512


def _caus_body(rel_ref, out_ref):
    x = rel_ref[0]
    rows = jax.lax.broadcasted_iota(jnp.int32, (S, 4 * S), 0)
    cols = jax.lax.broadcasted_iota(jnp.int32, (S, 4 * S), 1)
    hit = (x > 0.5) & ((cols >> 2) < rows)
    out_ref[0] = jnp.any(hit, axis=1).astype(jnp.float32)[:, None]


def _compute_caus(rel):
    rel3 = rel.reshape(B, S, 4 * S)
    out = pl.pallas_call(
        _caus_body,
        grid=(B,),
        in_specs=[pl.BlockSpec((1, S, 4 * S), lambda b: (b, 0, 0))],
        out_specs=pl.BlockSpec((1, S, 1), lambda b: (b, 0, 0)),
        out_shape=jax.ShapeDtypeStruct((B, S, 1), jnp.float32),
    )(rel3)
    return out.reshape(B, S)


def _shift_fwd(x, d, fill, lane):
    r = pltpu.roll(x, d, 1)
    return jnp.where(lane >= d, r, fill)


def _shift_bwd(x, d, fill, lane):
    r = pltpu.roll(x, S - d, 1)
    return jnp.where(lane < S - d, r, fill)


def _main_body(xt_ref, at_ref, ot_ref, ea_ref, eo_ref, caus_ref,
               w1_ref, b1_ref, w2_ref, b2_ref, w3_ref, b3_ref,
               ca_ref, co_ref):
    f32 = jnp.float32

    w1, w2, w3 = w1_ref[...], w2_ref[...], w3_ref[...]
    b1, b2, b3 = b1_ref[...], b2_ref[...], b3_ref[...]
    rows = []
    for b in range(B):
        xb = xt_ref[b]
        h = jnp.maximum(jnp.dot(w1, xb, preferred_element_type=f32) + b1, 0.0)
        h = jnp.maximum(jnp.dot(w2, h, preferred_element_type=f32) + b2, 0.0)
        z = jnp.dot(w3, h, preferred_element_type=f32) + b3
        rows.append(z)
    z = jnp.concatenate(rows, axis=0)
    score = jax.nn.sigmoid(z)
    f1 = jnp.where(score < 0.5, 2.0 * score, 1.0)

    a0, a1, a2, a3 = at_ref[0], at_ref[1], at_ref[2], at_ref[3]
    o0, o1, o2, o3 = ot_ref[0], ot_ref[1], ot_ref[2], ot_ref[3]

    def imp(c0, c1, c2, c3):
        m = jnp.maximum(jnp.maximum(c0, c1), jnp.maximum(c2, c3))
        e0, e1 = jnp.exp(c0 - m), jnp.exp(c1 - m)
        e2, e3 = jnp.exp(c2 - m), jnp.exp(c3 - m)
        den = ((e0 + e1) + e2) + e3
        return (e0 / den + e1 / den) > 0.5

    imp_asp = imp(a0, a1, a2, a3)
    imp_op = imp(o0, o1, o2, o3)

    ri = jax.lax.broadcasted_iota(jnp.int32, (S, S), 0)
    ci = jax.lax.broadcasted_iota(jnp.int32, (S, S), 1)
    band = (jnp.abs(ri - ci) <= 3).astype(f32)
    ea = (ea_ref[...] > 0).astype(f32)
    eo = (eo_ref[...] > 0).astype(f32)
    near_op = jnp.dot(eo, band, preferred_element_type=f32) > 0.0
    near_as = jnp.dot(ea, band, preferred_element_type=f32) > 0.0

    r2 = imp_asp & (~near_op)
    r3 = imp_op & (~near_as)
    caus = caus_ref[...] > 0.5

    w2a = jnp.where(r2, 0.3, 1.0)
    w2o = jnp.where(r3, 0.3, 1.0)
    w7 = jnp.where(caus, 0.7, 1.0)
    u_a0, u_a1 = (a0 * f1) * w2a, (a1 * f1) * w2a
    u_o0, u_o1 = (o0 * f1) * w2o, (o1 * f1) * w2o
    t_a2, t_a3 = a2 * f1, a3 * f1
    t_o2, t_o3 = o2 * f1, o3 * f1

    P = jnp.maximum(jnp.maximum(t_a2, t_a3), jnp.maximum(t_o2, t_o3)) > 0.5
    Q1 = jnp.maximum(jnp.maximum(u_a0 * w7, u_a1 * w7),
                     jnp.maximum(u_o0, u_o1)) > 0.5
    Q0 = jnp.maximum(jnp.maximum((u_a0 * 0.1) * w7, (u_a1 * 0.1) * w7),
                     jnp.maximum(u_o0 * 0.1, u_o1 * 0.1)) > 0.5
    F = jnp.maximum(
        jnp.maximum(jnp.maximum(a0, a1), jnp.maximum(a2, a3)),
        jnp.maximum(jnp.maximum(o0, o1), jnp.maximum(o2, o3))) > 0.5

    Rf = (P | Q0).astype(f32)
    Q1f = Q1.astype(f32)
    Ff = F.astype(f32)

    lane = jax.lax.broadcasted_iota(jnp.int32, (B, S), 1)
    base = jnp.maximum(
        jnp.maximum(_shift_bwd(Ff, 1, 0.0, lane), _shift_bwd(Ff, 2, 0.0, lane)),
        jnp.maximum(_shift_fwd(Rf, 1, 0.0, lane), _shift_fwd(Rf, 2, 0.0, lane)))
    q1s1 = _shift_fwd(Q1f, 1, 0.0, lane)
    q1s2 = _shift_fwd(Q1f, 2, 0.0, lane)

    ones = jnp.ones((B, S), f32)
    zeros = jnp.zeros((B, S), f32)
    a11, a12, a21, a22 = q1s1, q1s2, ones, zeros
    c1, c2 = base, zeros
    d = 1
    while d < S:
        b11 = _shift_fwd(a11, d, 1.0, lane)
        b12 = _shift_fwd(a12, d, 0.0, lane)
        b21 = _shift_fwd(a21, d, 0.0, lane)
        b22 = _shift_fwd(a22, d, 1.0, lane)
        bc1 = _shift_fwd(c1, d, 0.0, lane)
        bc2 = _shift_fwd(c2, d, 0.0, lane)
        n11 = jnp.maximum(a11 * b11, a12 * b21)
        n12 = jnp.maximum(a11 * b12, a12 * b22)
        n21 = jnp.maximum(a21 * b11, a22 * b21)
        n22 = jnp.maximum(a21 * b12, a22 * b22)
        nc1 = jnp.maximum(jnp.maximum(a11 * bc1, a12 * bc2), c1)
        nc2 = jnp.maximum(jnp.maximum(a21 * bc1, a22 * bc2), c2)
        a11, a12, a21, a22, c1, c2 = n11, n12, n21, n22, nc1, nc2
        d *= 2

    act = c1
    iso = jnp.where(act > 0.5, 1.0, 0.1)

    ca_ref[0] = (u_a0 * iso) * w7
    ca_ref[1] = (u_a1 * iso) * w7
    ca_ref[2] = t_a2
    ca_ref[3] = t_a3
    co_ref[0] = u_o0 * iso
    co_ref[1] = u_o1 * iso
    co_ref[2] = t_o2
    co_ref[3] = t_o3


def kernel(aspect_logits, opinion_logits, aspect_opinion_relations,
           explicit_aspects, explicit_opinions, W1, b1, W2, b2, W3, b3):
    caus = _compute_caus(aspect_opinion_relations)

    at = jnp.transpose(aspect_logits, (2, 0, 1))
    ot = jnp.transpose(opinion_logits, (2, 0, 1))
    x = jnp.concatenate([aspect_logits, opinion_logits], axis=-1)
    xt = jnp.transpose(x, (0, 2, 1))
    ea = explicit_aspects.astype(jnp.int32)
    eo = explicit_opinions.astype(jnp.int32)
    w1t = W1.T
    w2t = W2.T
    w3t = W3.T
    b1c = b1.reshape(32, 1)
    b2c = b2.reshape(16, 1)
    b3c = b3.reshape(1, 1)

    ca_t, co_t = pl.pallas_call(
        _main_body,
        out_shape=(jax.ShapeDtypeStruct((4, B, S), jnp.float32),
                   jax.ShapeDtypeStruct((4, B, S), jnp.float32)),
    )(xt, at, ot, ea, eo, caus, w1t, b1c, w2t, b2c, w3t, b3c)

    ca = jnp.transpose(ca_t, (1, 2, 0))
    co = jnp.transpose(co_t, (1, 2, 0))
    return ca, co


# speedup vs baseline: 93.0383x; 3.1379x over previous
import functools

import jax
import jax.numpy as jnp
from jax.experimental import pallas as pl
from jax.experimental.pallas import tpu as pltpu

B, S = 8, 512


def _caus_body(rel_ref, out_ref):
    x = rel_ref[0]
    i_iota = jax.lax.broadcasted_iota(jnp.int32, (S, 16, 128), 0)
    m_iota = jax.lax.broadcasted_iota(jnp.int32, (S, 16, 128), 1)
    l_iota = jax.lax.broadcasted_iota(jnp.int32, (S, 16, 128), 2)
    j = ((m_iota >> 2) << 7) + l_iota
    hit = (x > 0.5) & (j < i_iota)
    out_ref[0] = jnp.any(hit, axis=(1, 2)).astype(jnp.float32)[:, None]


def _compute_caus(rel):
    rel_v = (rel.reshape(B, S, 4, 128, 4)
             .transpose(0, 1, 2, 4, 3)
             .reshape(B, S, 16, 128))
    out = pl.pallas_call(
        _caus_body,
        grid=(B,),
        in_specs=[pl.BlockSpec((1, S, 16, 128), lambda b: (b, 0, 0, 0))],
        out_specs=pl.BlockSpec((1, S, 1), lambda b: (b, 0, 0)),
        out_shape=jax.ShapeDtypeStruct((B, S, 1), jnp.float32),
    )(rel_v)
    return out.reshape(B, S)


def _shift_fwd(x, d, fill, lane):
    r = pltpu.roll(x, d, 1)
    return jnp.where(lane >= d, r, fill)


def _shift_bwd(x, d, fill, lane):
    r = pltpu.roll(x, S - d, 1)
    return jnp.where(lane < S - d, r, fill)


def _main_body(xt_ref, at_ref, ot_ref, ea_ref, eo_ref, caus_ref,
               w1_ref, b1_ref, w2_ref, b2_ref, w3_ref, b3_ref,
               ca_ref, co_ref):
    f32 = jnp.float32

    w1, w2, w3 = w1_ref[...], w2_ref[...], w3_ref[...]
    b1, b2, b3 = b1_ref[...], b2_ref[...], b3_ref[...]
    rows = []
    for b in range(B):
        xb = xt_ref[b]
        h = jnp.maximum(jnp.dot(w1, xb, preferred_element_type=f32) + b1, 0.0)
        h = jnp.maximum(jnp.dot(w2, h, preferred_element_type=f32) + b2, 0.0)
        z = jnp.dot(w3, h, preferred_element_type=f32) + b3
        rows.append(z)
    z = jnp.concatenate(rows, axis=0)
    score = jax.nn.sigmoid(z)
    f1 = jnp.where(score < 0.5, 2.0 * score, 1.0)

    a0, a1, a2, a3 = at_ref[0], at_ref[1], at_ref[2], at_ref[3]
    o0, o1, o2, o3 = ot_ref[0], ot_ref[1], ot_ref[2], ot_ref[3]

    def imp(c0, c1, c2, c3):
        m = jnp.maximum(jnp.maximum(c0, c1), jnp.maximum(c2, c3))
        e0, e1 = jnp.exp(c0 - m), jnp.exp(c1 - m)
        e2, e3 = jnp.exp(c2 - m), jnp.exp(c3 - m)
        den = ((e0 + e1) + e2) + e3
        return (e0 / den + e1 / den) > 0.5

    imp_asp = imp(a0, a1, a2, a3)
    imp_op = imp(o0, o1, o2, o3)

    ri = jax.lax.broadcasted_iota(jnp.int32, (S, S), 0)
    ci = jax.lax.broadcasted_iota(jnp.int32, (S, S), 1)
    band = (jnp.abs(ri - ci) <= 3).astype(f32)
    ea = (ea_ref[...] > 0).astype(f32)
    eo = (eo_ref[...] > 0).astype(f32)
    near_op = jnp.dot(eo, band, preferred_element_type=f32) > 0.0
    near_as = jnp.dot(ea, band, preferred_element_type=f32) > 0.0

    r2 = imp_asp & (~near_op)
    r3 = imp_op & (~near_as)
    caus = caus_ref[...] > 0.5

    w2a = jnp.where(r2, 0.3, 1.0)
    w2o = jnp.where(r3, 0.3, 1.0)
    w7 = jnp.where(caus, 0.7, 1.0)
    u_a0, u_a1 = (a0 * f1) * w2a, (a1 * f1) * w2a
    u_o0, u_o1 = (o0 * f1) * w2o, (o1 * f1) * w2o
    t_a2, t_a3 = a2 * f1, a3 * f1
    t_o2, t_o3 = o2 * f1, o3 * f1

    P = jnp.maximum(jnp.maximum(t_a2, t_a3), jnp.maximum(t_o2, t_o3)) > 0.5
    Q1 = jnp.maximum(jnp.maximum(u_a0 * w7, u_a1 * w7),
                     jnp.maximum(u_o0, u_o1)) > 0.5
    Q0 = jnp.maximum(jnp.maximum((u_a0 * 0.1) * w7, (u_a1 * 0.1) * w7),
                     jnp.maximum(u_o0 * 0.1, u_o1 * 0.1)) > 0.5
    F = jnp.maximum(
        jnp.maximum(jnp.maximum(a0, a1), jnp.maximum(a2, a3)),
        jnp.maximum(jnp.maximum(o0, o1), jnp.maximum(o2, o3))) > 0.5

    Rf = (P | Q0).astype(f32)
    Q1f = Q1.astype(f32)
    Ff = F.astype(f32)

    lane = jax.lax.broadcasted_iota(jnp.int32, (B, S), 1)
    base = jnp.maximum(
        jnp.maximum(_shift_bwd(Ff, 1, 0.0, lane), _shift_bwd(Ff, 2, 0.0, lane)),
        jnp.maximum(_shift_fwd(Rf, 1, 0.0, lane), _shift_fwd(Rf, 2, 0.0, lane)))
    q1s1 = _shift_fwd(Q1f, 1, 0.0, lane)
    q1s2 = _shift_fwd(Q1f, 2, 0.0, lane)

    ones = jnp.ones((B, S), f32)
    zeros = jnp.zeros((B, S), f32)
    a11, a12, a21, a22 = q1s1, q1s2, ones, zeros
    c1, c2 = base, zeros
    d = 1
    while d < S:
        b11 = _shift_fwd(a11, d, 1.0, lane)
        b12 = _shift_fwd(a12, d, 0.0, lane)
        b21 = _shift_fwd(a21, d, 0.0, lane)
        b22 = _shift_fwd(a22, d, 1.0, lane)
        bc1 = _shift_fwd(c1, d, 0.0, lane)
        bc2 = _shift_fwd(c2, d, 0.0, lane)
        n11 = jnp.maximum(a11 * b11, a12 * b21)
        n12 = jnp.maximum(a11 * b12, a12 * b22)
        n21 = jnp.maximum(a21 * b11, a22 * b21)
        n22 = jnp.maximum(a21 * b12, a22 * b22)
        nc1 = jnp.maximum(jnp.maximum(a11 * bc1, a12 * bc2), c1)
        nc2 = jnp.maximum(jnp.maximum(a21 * bc1, a22 * bc2), c2)
        a11, a12, a21, a22, c1, c2 = n11, n12, n21, n22, nc1, nc2
        d *= 2

    act = c1
    iso = jnp.where(act > 0.5, 1.0, 0.1)

    ca_ref[0] = (u_a0 * iso) * w7
    ca_ref[1] = (u_a1 * iso) * w7
    ca_ref[2] = t_a2
    ca_ref[3] = t_a3
    co_ref[0] = u_o0 * iso
    co_ref[1] = u_o1 * iso
    co_ref[2] = t_o2
    co_ref[3] = t_o3


def kernel(aspect_logits, opinion_logits, aspect_opinion_relations,
           explicit_aspects, explicit_opinions, W1, b1, W2, b2, W3, b3):
    caus = _compute_caus(aspect_opinion_relations)

    at = jnp.transpose(aspect_logits, (2, 0, 1))
    ot = jnp.transpose(opinion_logits, (2, 0, 1))
    x = jnp.concatenate([aspect_logits, opinion_logits], axis=-1)
    xt = jnp.transpose(x, (0, 2, 1))
    ea = explicit_aspects.astype(jnp.int32)
    eo = explicit_opinions.astype(jnp.int32)
    w1t = W1.T
    w2t = W2.T
    w3t = W3.T
    b1c = b1.reshape(32, 1)
    b2c = b2.reshape(16, 1)
    b3c = b3.reshape(1, 1)

    ca_t, co_t = pl.pallas_call(
        _main_body,
        out_shape=(jax.ShapeDtypeStruct((4, B, S), jnp.float32),
                   jax.ShapeDtypeStruct((4, B, S), jnp.float32)),
    )(xt, at, ot, ea, eo, caus, w1t, b1c, w2t, b2c, w3t, b3c)

    ca = jnp.transpose(ca_t, (1, 2, 0))
    co = jnp.transpose(co_t, (1, 2, 0))
    return ca, co


# speedup vs baseline: 132.6162x; 1.4254x over previous
import jax
import jax.numpy as jnp
from jax.experimental import pallas as pl
from jax.experimental.pallas import tpu as pltpu

B, S = 8, 512
F32 = jnp.float32


def _shift_fwd(x, d, fill, lane):
    r = pltpu.roll(x, d, 1)
    return jnp.where(lane >= d, r, fill)


def _shift_bwd(x, d, fill, lane):
    r = pltpu.roll(x, S - d, 1)
    return jnp.where(lane < S - d, r, fill)


def _channels(view):
    return [jnp.concatenate([view[:, 4 * k + c, :] for k in range(4)], axis=1)
            for c in range(4)]


def _body(rel_ref, av_ref, ov_ref, ea_ref, eo_ref,
          w1t_ref, b1c_ref, w2t_ref, b2c_ref, w3t_ref, b3c_ref,
          cav_ref, cov_ref, caus_sc):
    b = pl.program_id(0)

    x = rel_ref[0]
    i_iota = jax.lax.broadcasted_iota(jnp.int32, (S, 16, 128), 0)
    m_iota = jax.lax.broadcasted_iota(jnp.int32, (S, 16, 128), 1)
    l_iota = jax.lax.broadcasted_iota(jnp.int32, (S, 16, 128), 2)
    j = ((m_iota >> 2) << 7) + l_iota
    hit = (x > 0.5) & (j < i_iota)
    caus_sc[b] = jnp.any(hit, axis=(1, 2)).astype(F32)[:, None]

    @pl.when(b == B - 1)
    def _main():
        a0, a1, a2, a3 = _channels(av_ref[...])
        o0, o1, o2, o3 = _channels(ov_ref[...])

        feats = [a0, a1, a2, a3, o0, o1, o2, o3]
        xall = jnp.concatenate(
            [jnp.reshape(f, (1, B * S)) for f in feats], axis=0)
        h = jnp.dot(w1t_ref[...], xall, preferred_element_type=F32) + b1c_ref[...]
        h = jnp.maximum(h, 0.0)
        h = jnp.dot(w2t_ref[...], h, preferred_element_type=F32) + b2c_ref[...]
        h = jnp.maximum(h, 0.0)
        z = jnp.dot(w3t_ref[...], h, preferred_element_type=F32) + b3c_ref[...]
        score = jax.nn.sigmoid(jnp.reshape(z, (B, S)))
        f1 = jnp.where(score < 0.5, 2.0 * score, 1.0)

        def imp(c0, c1, c2, c3):
            m = jnp.maximum(jnp.maximum(c0, c1), jnp.maximum(c2, c3))
            e0, e1 = jnp.exp(c0 - m), jnp.exp(c1 - m)
            e2, e3 = jnp.exp(c2 - m), jnp.exp(c3 - m)
            den = ((e0 + e1) + e2) + e3
            return (e0 / den + e1 / den) > 0.5

        imp_asp = imp(a0, a1, a2, a3)
        imp_op = imp(o0, o1, o2, o3)

        ri = jax.lax.broadcasted_iota(jnp.int32, (S, S), 0)
        ci = jax.lax.broadcasted_iota(jnp.int32, (S, S), 1)
        band = (jnp.abs(ri - ci) <= 3).astype(F32)
        ea = (ea_ref[...] > 0).astype(F32)
        eo = (eo_ref[...] > 0).astype(F32)
        near_op = jnp.dot(eo, band, preferred_element_type=F32) > 0.0
        near_as = jnp.dot(ea, band, preferred_element_type=F32) > 0.0

        r2 = imp_asp & (~near_op)
        r3 = imp_op & (~near_as)
        caus = jnp.reshape(caus_sc[...], (B, S)) > 0.5

        w2a = jnp.where(r2, 0.3, 1.0)
        w2o = jnp.where(r3, 0.3, 1.0)
        w7 = jnp.where(caus, 0.7, 1.0)
        u_a0, u_a1 = (a0 * f1) * w2a, (a1 * f1) * w2a
        u_o0, u_o1 = (o0 * f1) * w2o, (o1 * f1) * w2o
        t_a2, t_a3 = a2 * f1, a3 * f1
        t_o2, t_o3 = o2 * f1, o3 * f1

        P = jnp.maximum(jnp.maximum(t_a2, t_a3), jnp.maximum(t_o2, t_o3)) > 0.5
        Q1 = jnp.maximum(jnp.maximum(u_a0 * w7, u_a1 * w7),
                         jnp.maximum(u_o0, u_o1)) > 0.5
        Q0 = jnp.maximum(jnp.maximum((u_a0 * 0.1) * w7, (u_a1 * 0.1) * w7),
                         jnp.maximum(u_o0 * 0.1, u_o1 * 0.1)) > 0.5
        F = jnp.maximum(
            jnp.maximum(jnp.maximum(a0, a1), jnp.maximum(a2, a3)),
            jnp.maximum(jnp.maximum(o0, o1), jnp.maximum(o2, o3))) > 0.5

        Rf = (P | Q0).astype(F32)
        Q1f = Q1.astype(F32)
        Ff = F.astype(F32)

        lane = jax.lax.broadcasted_iota(jnp.int32, (B, S), 1)
        base = jnp.maximum(
            jnp.maximum(_shift_bwd(Ff, 1, 0.0, lane),
                        _shift_bwd(Ff, 2, 0.0, lane)),
            jnp.maximum(_shift_fwd(Rf, 1, 0.0, lane),
                        _shift_fwd(Rf, 2, 0.0, lane)))
        q1s1 = _shift_fwd(Q1f, 1, 0.0, lane)
        q1s2 = _shift_fwd(Q1f, 2, 0.0, lane)

        ones = jnp.ones((B, S), F32)
        zeros = jnp.zeros((B, S), F32)
        a11, a12, a21, a22 = q1s1, q1s2, ones, zeros
        c1, c2 = base, zeros
        d = 1
        while d < S:
            b11 = _shift_fwd(a11, d, 1.0, lane)
            b12 = _shift_fwd(a12, d, 0.0, lane)
            b21 = _shift_fwd(a21, d, 0.0, lane)
            b22 = _shift_fwd(a22, d, 1.0, lane)
            bc1 = _shift_fwd(c1, d, 0.0, lane)
            bc2 = _shift_fwd(c2, d, 0.0, lane)
            n11 = jnp.maximum(a11 * b11, a12 * b21)
            n12 = jnp.maximum(a11 * b12, a12 * b22)
            n21 = jnp.maximum(a21 * b11, a22 * b21)
            n22 = jnp.maximum(a21 * b12, a22 * b22)
            nc1 = jnp.maximum(jnp.maximum(a11 * bc1, a12 * bc2), c1)
            nc2 = jnp.maximum(jnp.maximum(a21 * bc1, a22 * bc2), c2)
            a11, a12, a21, a22, c1, c2 = n11, n12, n21, n22, nc1, nc2
            d *= 2

        iso = jnp.where(c1 > 0.5, 1.0, 0.1)

        outs_a = ((u_a0 * iso) * w7, (u_a1 * iso) * w7, t_a2, t_a3)
        outs_o = (u_o0 * iso, u_o1 * iso, t_o2, t_o3)
        for k in range(4):
            sl = slice(128 * k, 128 * (k + 1))
            for c in range(4):
                cav_ref[:, 4 * k + c, :] = outs_a[c][:, sl]
                cov_ref[:, 4 * k + c, :] = outs_o[c][:, sl]


def _to_view(x):
    return (x.reshape(B, 4, 128, 4)
            .transpose(0, 1, 3, 2)
            .reshape(B, 16, 128))


def _from_view(v):
    return (v.reshape(B, 4, 4, 128)
            .transpose(0, 1, 3, 2)
            .reshape(B, S, 4))


def kernel(aspect_logits, opinion_logits, aspect_opinion_relations,
           explicit_aspects, explicit_opinions, W1, b1, W2, b2, W3, b3):
    rel_v = (aspect_opinion_relations.reshape(B, S, 4, 128, 4)
             .transpose(0, 1, 2, 4, 3)
             .reshape(B, S, 16, 128))
    av = _to_view(aspect_logits)
    ov = _to_view(opinion_logits)
    ea = explicit_aspects.astype(jnp.int32)
    eo = explicit_opinions.astype(jnp.int32)
    w1t = W1.T
    w2t = W2.T
    w3t = W3.T
    b1c = b1.reshape(32, 1)
    b2c = b2.reshape(16, 1)
    b3c = b3.reshape(1, 1)

    full = lambda shape: pl.BlockSpec(shape, lambda b: (0,) * len(shape))
    cav, cov = pl.pallas_call(
        _body,
        grid=(B,),
        in_specs=[
            pl.BlockSpec((1, S, 16, 128), lambda b: (b, 0, 0, 0)),
            full((B, 16, 128)), full((B, 16, 128)),
            full((B, S)), full((B, S)),
            full((32, 8)), full((32, 1)),
            full((16, 32)), full((16, 1)),
            full((1, 16)), full((1, 1)),
        ],
        out_specs=(full((B, 16, 128)), full((B, 16, 128))),
        out_shape=(jax.ShapeDtypeStruct((B, 16, 128), F32),
                   jax.ShapeDtypeStruct((B, 16, 128), F32)),
        scratch_shapes=[pltpu.VMEM((B, S, 1), F32)],
    )(rel_v, av, ov, ea, eo, w1t, b1c, w2t, b2c, w3t, b3c)

    return _from_view(cav), _from_view(cov)
